# baseline (speedup 1.0000x reference)
import jax
import jax.numpy as jnp
from jax import lax
from jax.experimental import pallas as pl
from jax.experimental.pallas import tpu as pltpu
from jax.experimental.pallas import tpu_sc as plsc

B = 4
NQ = 1024
C = 768
H = 32
W = 32
NH = 12
NP = 4
HD = 64
BH = B * NH
NCORN = NP * 4
QBLK = 128
NQB = NQ // QBLK
NC = 2
NS = 16
NW = NC * NS
QH = 512
TASKS = BH * (NQ // QH)
TPW = TASKS // NW


def _locs_body(q_ref, rp_ref, v_ref, w_ref, b_ref, wv_ref, bv_ref,
               pos_ref, wgt_ref, valt_ref):
    vt = lax.dot_general(wv_ref[...], v_ref[0], (((1,), (1,)), ((), ())),
                         preferred_element_type=jnp.float32)
    valt_ref[...] = vt.reshape(NH, HD, QBLK) + bv_ref[...]
    q = q_ref[0]
    proj = lax.dot_general(q, w_ref[...], (((1,), (1,)), ((), ())),
                           preferred_element_type=jnp.float32) + b_ref[0, 0]
    offx = proj[:, 0:48]
    offy = proj[:, 48:96]
    logits = proj[:, 96:144]
    rpx = rp_ref[0][:, 0:1]
    rpy = rp_ref[0][:, 1:2]
    a = [logits[:, p * NH:(p + 1) * NH] for p in range(NP)]
    m = jnp.maximum(jnp.maximum(a[0], a[1]), jnp.maximum(a[2], a[3]))
    e = [jnp.exp(v - m) for v in a]
    s = e[0] + e[1] + e[2] + e[3]
    x = jnp.clip(rpx + offx * (0.1 / W), 0.0, 1.0) * W - 0.5
    y = jnp.clip(rpy + offy * (0.1 / H), 0.0, 1.0) * H - 0.5
    x0 = jnp.floor(x)
    y0 = jnp.floor(y)
    fx = x - x0
    fy = y - y0
    for p in range(NP):
        attn_p = e[p] / s
        sl = slice(p * NH, (p + 1) * NH)
        x0p, y0p = x0[:, sl], y0[:, sl]
        fxp, fyp = fx[:, sl], fy[:, sl]
        for ci, (dx, dy) in enumerate(((0, 0), (0, 1), (1, 0), (1, 1))):
            xc = x0p + dx
            yc = y0p + dy
            wx = fxp if dx else 1.0 - fxp
            wy = fyp if dy else 1.0 - fyp
            valid = (xc >= 0) & (xc <= W - 1) & (yc >= 0) & (yc <= H - 1)
            wgt = wx * wy * attn_p * valid.astype(jnp.float32)
            posf = jnp.clip(yc, 0.0, H - 1) * W + jnp.clip(xc, 0.0, W - 1)
            i = ci * NP + p
            pos_ref[0, 0, :, i, :] = jnp.transpose(posf).astype(jnp.int32)
            wgt_ref[0, 0, :, i, :] = jnp.transpose(wgt)


def _compute_locs(query, reference_points, value, W_cat, b_cat, W_val, b_val):
    return pl.pallas_call(
        _locs_body,
        grid=(B, NQB),
        in_specs=[
            pl.BlockSpec((1, QBLK, C), lambda b, qb: (b, qb, 0)),
            pl.BlockSpec((1, QBLK, 2), lambda b, qb: (b, qb, 0)),
            pl.BlockSpec((1, QBLK, C), lambda b, qb: (b, qb, 0)),
            pl.BlockSpec((3 * NP * NH, C), lambda b, qb: (0, 0)),
            pl.BlockSpec((1, 1, 3 * NP * NH), lambda b, qb: (0, 0, 0)),
            pl.BlockSpec((C, C), lambda b, qb: (0, 0)),
            pl.BlockSpec((NH, HD, 1), lambda b, qb: (0, 0, 0)),
        ],
        out_specs=[
            pl.BlockSpec((1, 1, NH, NCORN, QBLK),
                         lambda b, qb: (qb // 4, b, 0, 0, qb % 4)),
            pl.BlockSpec((1, 1, NH, NCORN, QBLK),
                         lambda b, qb: (qb // 4, b, 0, 0, qb % 4)),
            pl.BlockSpec((NH, HD, QBLK), lambda b, qb: (b, 0, qb)),
        ],
        out_shape=[
            jax.ShapeDtypeStruct((2, B, NH, NCORN, QH), jnp.int32),
            jax.ShapeDtypeStruct((2, B, NH, NCORN, QH), jnp.float32),
            jax.ShapeDtypeStruct((BH, HD, H * W), jnp.float32),
        ],
    )(query, reference_points, value, W_cat, b_cat,
      W_val, b_val.reshape(NH, HD, 1))


def _sc_body(valt, pos, wgt, out, tbl, posv, wv, outv):
    wid = lax.axis_index("s") * NC + lax.axis_index("c")
    for t in range(TPW):
        task = wid * TPW + t
        bh = task % BH
        pltpu.sync_copy(valt.at[bh], tbl)
        pltpu.sync_copy(pos.at[task], posv)
        pltpu.sync_copy(wgt.at[task], wv)

        def qblock(j, carry):
            def dchunk(dc, carry2):
                accs = [jnp.zeros((16,), jnp.float32) for _ in range(16)]
                for i in range(NCORN):
                    pv = posv[pl.ds(i * QH + j * 16, 16)]
                    wvv = wv[pl.ds(i * QH + j * 16, 16)]
                    pvb = pv + dc * (16 * H * W)
                    for d in range(16):
                        g = plsc.load_gather(tbl, [pvb + d * (H * W)])
                        accs[d] = accs[d] + wvv * g
                for d in range(16):
                    outv[pl.ds((dc * 16 + d) * QH + j * 16, 16)] = accs[d]
                return carry2
            return lax.fori_loop(0, HD // 16, dchunk, carry, unroll=True)

        lax.fori_loop(0, QH // 16, qblock, 0)
        pltpu.sync_copy(outv, out.at[task])


def _sc_sample(valt, pos, wgt):
    return pl.kernel(
        _sc_body,
        out_type=jax.ShapeDtypeStruct((TASKS, QH * HD), jnp.float32),
        mesh=plsc.VectorSubcoreMesh(core_axis_name="c", subcore_axis_name="s"),
        compiler_params=pltpu.CompilerParams(needs_layout_passes=False),
        scratch_types=[
            pltpu.VMEM((HD * H * W,), jnp.float32),
            pltpu.VMEM((NCORN * QH,), jnp.int32),
            pltpu.VMEM((NCORN * QH,), jnp.float32),
            pltpu.VMEM((HD * QH,), jnp.float32),
        ],
    )(valt, pos, wgt)


def _outproj_body(f_ref, w_ref, b_ref, o_ref):
    h2 = pl.program_id(2)

    @pl.when(h2 == 0)
    def _():
        o_ref[0] = jnp.broadcast_to(b_ref[0], (QH, C))

    o_ref[0] = o_ref[0] + lax.dot_general(
        f_ref[0, 0, 0], w_ref[...], (((0,), (1,)), ((), ())),
        preferred_element_type=jnp.float32)


def _out_proj(feats, W_out, b_out):
    return pl.pallas_call(
        _outproj_body,
        grid=(B, 2, NH // 2),
        in_specs=[
            pl.BlockSpec((1, 1, 1, 2 * HD, QH),
                         lambda b, s, h2: (s, b, h2, 0, 0)),
            pl.BlockSpec((C, 2 * HD), lambda b, s, h2: (0, h2)),
            pl.BlockSpec((1, 1, C), lambda b, s, h2: (0, 0, 0)),
        ],
        out_specs=pl.BlockSpec((1, QH, C), lambda b, s, h2: (b, s, 0)),
        out_shape=jax.ShapeDtypeStruct((B, NQ, C), jnp.float32),
    )(feats, W_out, b_out.reshape(1, 1, C))


def kernel(query, reference_points, value, spatial_shapes,
           W_off, b_off, W_attn, b_attn, W_val, b_val, W_out, b_out):
    Wo = W_off.reshape(NH, NP, 2, C)
    bo = b_off.reshape(NH, NP, 2)
    Wx = jnp.transpose(Wo[:, :, 0, :], (1, 0, 2)).reshape(NP * NH, C)
    Wy = jnp.transpose(Wo[:, :, 1, :], (1, 0, 2)).reshape(NP * NH, C)
    Wa = jnp.transpose(W_attn.reshape(NH, NP, C), (1, 0, 2)).reshape(NP * NH, C)
    bx = jnp.transpose(bo[:, :, 0]).reshape(NP * NH)
    by = jnp.transpose(bo[:, :, 1]).reshape(NP * NH)
    ba = jnp.transpose(b_attn.reshape(NH, NP)).reshape(NP * NH)
    W_cat = jnp.concatenate([Wx, Wy, Wa], axis=0)
    b_cat = jnp.concatenate([bx, by, ba], axis=0).reshape(1, 1, 3 * NP * NH)

    pos, wgt, valt = _compute_locs(query, reference_points, value,
                                   W_cat, b_cat, W_val, b_val)
    feats = _sc_sample(valt.reshape(BH, HD * H * W),
                       pos.reshape(TASKS, NCORN * QH),
                       wgt.reshape(TASKS, NCORN * QH))
    feats = feats.reshape(2, B, NH // 2, 2 * HD, QH)
    return _out_proj(feats, W_out, b_out)


# speedup vs baseline: 1.1774x; 1.1774x over previous
import jax
import jax.numpy as jnp
from jax import lax
from jax.experimental import pallas as pl
from jax.experimental.pallas import tpu as pltpu
from jax.experimental.pallas import tpu_sc as plsc

B = 4
NQ = 1024
C = 768
H = 32
W = 32
NH = 12
NP = 4
HD = 64
BH = B * NH
NCORN = NP * 4
QBLK = 256
NQB = NQ // QBLK
NC = 2
NS = 16
NW = NC * NS
QH = 512
QPB = QH // QBLK
TASKS = BH * (NQ // QH)
TPW = TASKS // NW


def _locs_body(q_ref, rp_ref, v_ref, w_ref, b_ref, wv_ref, bv_ref,
               pos_ref, wgt_ref, valt_ref):
    vt = lax.dot_general(wv_ref[...], v_ref[0], (((1,), (1,)), ((), ())),
                         preferred_element_type=jnp.float32)
    valt_ref[...] = vt.reshape(NH, HD, QBLK) + bv_ref[...]
    q = q_ref[0]
    proj = lax.dot_general(q, w_ref[...], (((1,), (1,)), ((), ())),
                           preferred_element_type=jnp.float32) + b_ref[0, 0]
    offx = proj[:, 0:48]
    offy = proj[:, 48:96]
    logits = proj[:, 96:144]
    rpx = rp_ref[0][:, 0:1]
    rpy = rp_ref[0][:, 1:2]
    a = [logits[:, p * NH:(p + 1) * NH] for p in range(NP)]
    m = jnp.maximum(jnp.maximum(a[0], a[1]), jnp.maximum(a[2], a[3]))
    e = [jnp.exp(v - m) for v in a]
    s = e[0] + e[1] + e[2] + e[3]
    x = jnp.clip(rpx + offx * (0.1 / W), 0.0, 1.0) * W - 0.5
    y = jnp.clip(rpy + offy * (0.1 / H), 0.0, 1.0) * H - 0.5
    x0 = jnp.floor(x)
    y0 = jnp.floor(y)
    fx = x - x0
    fy = y - y0
    for p in range(NP):
        attn_p = e[p] / s
        sl = slice(p * NH, (p + 1) * NH)
        x0p, y0p = x0[:, sl], y0[:, sl]
        fxp, fyp = fx[:, sl], fy[:, sl]
        for ci, (dx, dy) in enumerate(((0, 0), (0, 1), (1, 0), (1, 1))):
            xc = x0p + dx
            yc = y0p + dy
            wx = fxp if dx else 1.0 - fxp
            wy = fyp if dy else 1.0 - fyp
            valid = (xc >= 0) & (xc <= W - 1) & (yc >= 0) & (yc <= H - 1)
            wgt = wx * wy * attn_p * valid.astype(jnp.float32)
            posf = jnp.clip(yc, 0.0, H - 1) * W + jnp.clip(xc, 0.0, W - 1)
            i = ci * NP + p
            pos_ref[0, 0, :, i, :] = jnp.transpose(posf).astype(jnp.int32)
            wgt_ref[0, 0, :, i, :] = jnp.transpose(wgt)


def _compute_locs(query, reference_points, value, W_cat, b_cat, W_val, b_val):
    return pl.pallas_call(
        _locs_body,
        grid=(B, NQB),
        in_specs=[
            pl.BlockSpec((1, QBLK, C), lambda b, qb: (b, qb, 0)),
            pl.BlockSpec((1, QBLK, 2), lambda b, qb: (b, qb, 0)),
            pl.BlockSpec((1, QBLK, C), lambda b, qb: (b, qb, 0)),
            pl.BlockSpec((3 * NP * NH, C), lambda b, qb: (0, 0)),
            pl.BlockSpec((1, 1, 3 * NP * NH), lambda b, qb: (0, 0, 0)),
            pl.BlockSpec((C, C), lambda b, qb: (0, 0)),
            pl.BlockSpec((NH, HD, 1), lambda b, qb: (0, 0, 0)),
        ],
        out_specs=[
            pl.BlockSpec((1, 1, NH, NCORN, QBLK),
                         lambda b, qb: (qb // QPB, b, 0, 0, qb % QPB)),
            pl.BlockSpec((1, 1, NH, NCORN, QBLK),
                         lambda b, qb: (qb // QPB, b, 0, 0, qb % QPB)),
            pl.BlockSpec((NH, HD, QBLK), lambda b, qb: (b, 0, qb)),
        ],
        out_shape=[
            jax.ShapeDtypeStruct((2, B, NH, NCORN, QH), jnp.int32),
            jax.ShapeDtypeStruct((2, B, NH, NCORN, QH), jnp.float32),
            jax.ShapeDtypeStruct((BH, HD, H * W), jnp.float32),
        ],
    )(query, reference_points, value, W_cat, b_cat,
      W_val, b_val.reshape(NH, HD, 1))


def _sc_body(valt, pos, wgt, out, tbl, posv, wv, outv):
    wid = lax.axis_index("s") * NC + lax.axis_index("c")
    for t in range(TPW):
        task = wid * TPW + t
        bh = task % BH
        pltpu.sync_copy(valt.at[bh], tbl)
        pltpu.sync_copy(pos.at[task], posv)
        pltpu.sync_copy(wgt.at[task], wv)

        def qblock(j, carry):
            def dchunk(dc, carry2):
                accs = [jnp.zeros((16,), jnp.float32) for _ in range(16)]
                for i in range(NCORN):
                    pv = posv[pl.ds(i * QH + j * 16, 16)]
                    wvv = wv[pl.ds(i * QH + j * 16, 16)]
                    pvb = pv + dc * (16 * H * W)
                    for d in range(16):
                        g = plsc.load_gather(tbl, [pvb + d * (H * W)])
                        accs[d] = accs[d] + wvv * g
                for d in range(16):
                    outv[pl.ds((dc * 16 + d) * QH + j * 16, 16)] = accs[d]
                return carry2
            return lax.fori_loop(0, HD // 16, dchunk, carry, unroll=2)

        lax.fori_loop(0, QH // 16, qblock, 0)
        pltpu.sync_copy(outv, out.at[task])


def _sc_sample(valt, pos, wgt):
    return pl.kernel(
        _sc_body,
        out_type=jax.ShapeDtypeStruct((TASKS, QH * HD), jnp.float32),
        mesh=plsc.VectorSubcoreMesh(core_axis_name="c", subcore_axis_name="s"),
        compiler_params=pltpu.CompilerParams(needs_layout_passes=False),
        scratch_types=[
            pltpu.VMEM((HD * H * W,), jnp.float32),
            pltpu.VMEM((NCORN * QH,), jnp.int32),
            pltpu.VMEM((NCORN * QH,), jnp.float32),
            pltpu.VMEM((HD * QH,), jnp.float32),
        ],
    )(valt, pos, wgt)


def _outproj_body(f_ref, w_ref, b_ref, o_ref):
    h2 = pl.program_id(2)

    @pl.when(h2 == 0)
    def _():
        o_ref[0] = jnp.broadcast_to(b_ref[0], (QH, C))

    o_ref[0] = o_ref[0] + lax.dot_general(
        f_ref[0, 0, 0], w_ref[...], (((0,), (1,)), ((), ())),
        preferred_element_type=jnp.float32)


def _out_proj(feats, W_out, b_out):
    return pl.pallas_call(
        _outproj_body,
        grid=(B, 2, NH // 2),
        in_specs=[
            pl.BlockSpec((1, 1, 1, 2 * HD, QH),
                         lambda b, s, h2: (s, b, h2, 0, 0)),
            pl.BlockSpec((C, 2 * HD), lambda b, s, h2: (0, h2)),
            pl.BlockSpec((1, 1, C), lambda b, s, h2: (0, 0, 0)),
        ],
        out_specs=pl.BlockSpec((1, QH, C), lambda b, s, h2: (b, s, 0)),
        out_shape=jax.ShapeDtypeStruct((B, NQ, C), jnp.float32),
    )(feats, W_out, b_out.reshape(1, 1, C))


def kernel(query, reference_points, value, spatial_shapes,
           W_off, b_off, W_attn, b_attn, W_val, b_val, W_out, b_out):
    Wo = W_off.reshape(NH, NP, 2, C)
    bo = b_off.reshape(NH, NP, 2)
    Wx = jnp.transpose(Wo[:, :, 0, :], (1, 0, 2)).reshape(NP * NH, C)
    Wy = jnp.transpose(Wo[:, :, 1, :], (1, 0, 2)).reshape(NP * NH, C)
    Wa = jnp.transpose(W_attn.reshape(NH, NP, C), (1, 0, 2)).reshape(NP * NH, C)
    bx = jnp.transpose(bo[:, :, 0]).reshape(NP * NH)
    by = jnp.transpose(bo[:, :, 1]).reshape(NP * NH)
    ba = jnp.transpose(b_attn.reshape(NH, NP)).reshape(NP * NH)
    W_cat = jnp.concatenate([Wx, Wy, Wa], axis=0)
    b_cat = jnp.concatenate([bx, by, ba], axis=0).reshape(1, 1, 3 * NP * NH)

    pos, wgt, valt = _compute_locs(query, reference_points, value,
                                   W_cat, b_cat, W_val, b_val)
    feats = _sc_sample(valt.reshape(BH, HD * H * W),
                       pos.reshape(TASKS, NCORN * QH),
                       wgt.reshape(TASKS, NCORN * QH))
    feats = feats.reshape(2, B, NH // 2, 2 * HD, QH)
    return _out_proj(feats, W_out, b_out)


# speedup vs baseline: 1.1985x; 1.0180x over previous
import jax
import jax.numpy as jnp
from jax import lax
from jax.experimental import pallas as pl
from jax.experimental.pallas import tpu as pltpu
from jax.experimental.pallas import tpu_sc as plsc

B = 4
NQ = 1024
C = 768
H = 32
W = 32
NH = 12
NP = 4
HD = 64
BH = B * NH
NCORN = NP * 4
QBLK = 512
NQB = NQ // QBLK
NC = 2
NS = 16
NW = NC * NS
QH = 512
QPB = QH // QBLK
TASKS = BH * (NQ // QH)
TPW = TASKS // NW


def _locs_body(q_ref, rp_ref, v_ref, w_ref, b_ref, wv_ref, bv_ref,
               pos_ref, wgt_ref, valt_ref):
    vt = lax.dot_general(wv_ref[...], v_ref[0], (((1,), (1,)), ((), ())),
                         preferred_element_type=jnp.float32)
    valt_ref[...] = vt.reshape(NH, HD, QBLK) + bv_ref[...]
    q = q_ref[0]
    proj = lax.dot_general(q, w_ref[...], (((1,), (1,)), ((), ())),
                           preferred_element_type=jnp.float32) + b_ref[0, 0]
    offx = proj[:, 0:48]
    offy = proj[:, 48:96]
    logits = proj[:, 96:144]
    rpx = rp_ref[0][:, 0:1]
    rpy = rp_ref[0][:, 1:2]
    a = [logits[:, p * NH:(p + 1) * NH] for p in range(NP)]
    m = jnp.maximum(jnp.maximum(a[0], a[1]), jnp.maximum(a[2], a[3]))
    e = [jnp.exp(v - m) for v in a]
    s = e[0] + e[1] + e[2] + e[3]
    x = jnp.clip(rpx + offx * (0.1 / W), 0.0, 1.0) * W - 0.5
    y = jnp.clip(rpy + offy * (0.1 / H), 0.0, 1.0) * H - 0.5
    x0 = jnp.floor(x)
    y0 = jnp.floor(y)
    fx = x - x0
    fy = y - y0
    for p in range(NP):
        attn_p = e[p] / s
        sl = slice(p * NH, (p + 1) * NH)
        x0p, y0p = x0[:, sl], y0[:, sl]
        fxp, fyp = fx[:, sl], fy[:, sl]
        for ci, (dx, dy) in enumerate(((0, 0), (0, 1), (1, 0), (1, 1))):
            xc = x0p + dx
            yc = y0p + dy
            wx = fxp if dx else 1.0 - fxp
            wy = fyp if dy else 1.0 - fyp
            valid = (xc >= 0) & (xc <= W - 1) & (yc >= 0) & (yc <= H - 1)
            wgt = wx * wy * attn_p * valid.astype(jnp.float32)
            posf = jnp.clip(yc, 0.0, H - 1) * W + jnp.clip(xc, 0.0, W - 1)
            i = ci * NP + p
            pos_ref[0, 0, :, i, :] = jnp.transpose(posf).astype(jnp.int32)
            wgt_ref[0, 0, :, i, :] = jnp.transpose(wgt)


def _compute_locs(query, reference_points, value, W_cat, b_cat, W_val, b_val):
    return pl.pallas_call(
        _locs_body,
        grid=(B, NQB),
        in_specs=[
            pl.BlockSpec((1, QBLK, C), lambda b, qb: (b, qb, 0)),
            pl.BlockSpec((1, QBLK, 2), lambda b, qb: (b, qb, 0)),
            pl.BlockSpec((1, QBLK, C), lambda b, qb: (b, qb, 0)),
            pl.BlockSpec((3 * NP * NH, C), lambda b, qb: (0, 0)),
            pl.BlockSpec((1, 1, 3 * NP * NH), lambda b, qb: (0, 0, 0)),
            pl.BlockSpec((C, C), lambda b, qb: (0, 0)),
            pl.BlockSpec((NH, HD, 1), lambda b, qb: (0, 0, 0)),
        ],
        out_specs=[
            pl.BlockSpec((1, 1, NH, NCORN, QBLK),
                         lambda b, qb: (qb // QPB, b, 0, 0, qb % QPB)),
            pl.BlockSpec((1, 1, NH, NCORN, QBLK),
                         lambda b, qb: (qb // QPB, b, 0, 0, qb % QPB)),
            pl.BlockSpec((NH, HD, QBLK), lambda b, qb: (b, 0, qb)),
        ],
        out_shape=[
            jax.ShapeDtypeStruct((2, B, NH, NCORN, QH), jnp.int32),
            jax.ShapeDtypeStruct((2, B, NH, NCORN, QH), jnp.float32),
            jax.ShapeDtypeStruct((BH, HD, H * W), jnp.float32),
        ],
    )(query, reference_points, value, W_cat, b_cat,
      W_val, b_val.reshape(NH, HD, 1))


def _sc_body(valt, pos, wgt, out, tbl, posv, wv, outv):
    wid = lax.axis_index("s") * NC + lax.axis_index("c")
    for t in range(TPW):
        task = wid * TPW + t
        bh = task % BH
        pltpu.sync_copy(valt.at[bh], tbl)
        pltpu.sync_copy(pos.at[task], posv)
        pltpu.sync_copy(wgt.at[task], wv)

        def qblock(j, carry):
            def dchunk(dc, carry2):
                accs = [jnp.zeros((16,), jnp.float32) for _ in range(16)]
                for i in range(NCORN):
                    pv = posv[pl.ds(i * QH + j * 16, 16)]
                    wvv = wv[pl.ds(i * QH + j * 16, 16)]
                    pvb = pv + dc * (16 * H * W)
                    for d in range(16):
                        g = plsc.load_gather(tbl, [pvb + d * (H * W)])
                        accs[d] = accs[d] + wvv * g
                for d in range(16):
                    outv[pl.ds((dc * 16 + d) * QH + j * 16, 16)] = accs[d]
                return carry2
            return lax.fori_loop(0, HD // 16, dchunk, carry, unroll=2)

        lax.fori_loop(0, QH // 16, qblock, 0)
        pltpu.sync_copy(outv, out.at[task])


def _sc_sample(valt, pos, wgt):
    return pl.kernel(
        _sc_body,
        out_type=jax.ShapeDtypeStruct((TASKS, QH * HD), jnp.float32),
        mesh=plsc.VectorSubcoreMesh(core_axis_name="c", subcore_axis_name="s"),
        compiler_params=pltpu.CompilerParams(needs_layout_passes=False),
        scratch_types=[
            pltpu.VMEM((HD * H * W,), jnp.float32),
            pltpu.VMEM((NCORN * QH,), jnp.int32),
            pltpu.VMEM((NCORN * QH,), jnp.float32),
            pltpu.VMEM((HD * QH,), jnp.float32),
        ],
    )(valt, pos, wgt)


def _outproj_body(f_ref, w_ref, b_ref, o_ref):
    h2 = pl.program_id(2)

    @pl.when(h2 == 0)
    def _():
        o_ref[0] = jnp.broadcast_to(b_ref[0], (QH, C))

    o_ref[0] = o_ref[0] + lax.dot_general(
        f_ref[0, 0, 0], w_ref[...], (((0,), (1,)), ((), ())),
        preferred_element_type=jnp.float32)


def _out_proj(feats, W_out, b_out):
    return pl.pallas_call(
        _outproj_body,
        grid=(B, 2, NH // 2),
        in_specs=[
            pl.BlockSpec((1, 1, 1, 2 * HD, QH),
                         lambda b, s, h2: (s, b, h2, 0, 0)),
            pl.BlockSpec((C, 2 * HD), lambda b, s, h2: (0, h2)),
            pl.BlockSpec((1, 1, C), lambda b, s, h2: (0, 0, 0)),
        ],
        out_specs=pl.BlockSpec((1, QH, C), lambda b, s, h2: (b, s, 0)),
        out_shape=jax.ShapeDtypeStruct((B, NQ, C), jnp.float32),
    )(feats, W_out, b_out.reshape(1, 1, C))


def kernel(query, reference_points, value, spatial_shapes,
           W_off, b_off, W_attn, b_attn, W_val, b_val, W_out, b_out):
    Wo = W_off.reshape(NH, NP, 2, C)
    bo = b_off.reshape(NH, NP, 2)
    Wx = jnp.transpose(Wo[:, :, 0, :], (1, 0, 2)).reshape(NP * NH, C)
    Wy = jnp.transpose(Wo[:, :, 1, :], (1, 0, 2)).reshape(NP * NH, C)
    Wa = jnp.transpose(W_attn.reshape(NH, NP, C), (1, 0, 2)).reshape(NP * NH, C)
    bx = jnp.transpose(bo[:, :, 0]).reshape(NP * NH)
    by = jnp.transpose(bo[:, :, 1]).reshape(NP * NH)
    ba = jnp.transpose(b_attn.reshape(NH, NP)).reshape(NP * NH)
    W_cat = jnp.concatenate([Wx, Wy, Wa], axis=0)
    b_cat = jnp.concatenate([bx, by, ba], axis=0).reshape(1, 1, 3 * NP * NH)

    pos, wgt, valt = _compute_locs(query, reference_points, value,
                                   W_cat, b_cat, W_val, b_val)
    feats = _sc_sample(valt.reshape(BH, HD * H * W),
                       pos.reshape(TASKS, NCORN * QH),
                       wgt.reshape(TASKS, NCORN * QH))
    feats = feats.reshape(2, B, NH // 2, 2 * HD, QH)
    return _out_proj(feats, W_out, b_out)
